# baseline (speedup 1.0000x reference)
import functools

import numpy as np
import jax
import jax.numpy as jnp
from jax import lax
from jax.experimental import pallas as pl
from jax.experimental.pallas import tpu as pltpu
from jax.experimental.pallas import tpu_sc as plsc

F32 = jnp.float32
BF16 = jnp.bfloat16
H = 128
NSTEP = 50
EPS = 1e-5

SC_CORES = 2
SC_SUBCORES = 16
SC_WORKERS = SC_CORES * SC_SUBCORES


def _lrelu(x):
    return jnp.where(x >= 0, x, 0.01 * x)


_MASK_HI = -65536
_MASK_LO = 0xFFFF


def _rne_bf16_bits(f):
    b = lax.bitcast_convert_type(f, jnp.int32)
    return b + 0x7FFF + ((b >> 16) & 1)


def _pack_pair(lo, hi):
    return (((_rne_bf16_bits(lo) >> 16) & _MASK_LO)
            | (_rne_bf16_bits(hi) & _MASK_HI))


def _unpack_lo(p):
    return lax.bitcast_convert_type(p << 16, F32)


def _unpack_hi(p):
    return lax.bitcast_convert_type(p & _MASK_HI, F32)


def _prep_body(x_ref, w1a_ref, w1c_ref, wd_ref, bd_ref, tsrc_ref, u3_ref,
               xd_ref):
    xb = x_ref[...]
    xd = jnp.dot(xb, wd_ref[...], preferred_element_type=F32) + bd_ref[...]
    u1 = jnp.dot(xb, w1a_ref[...], preferred_element_type=F32)
    tsrc_ref[...] = _pack_pair(u1, xd)
    u3_ref[...] = jnp.dot(xb, w1c_ref[...], preferred_element_type=F32)
    xd_ref[...] = xd


def _prep(x, w1a, w1c, wd, bd, nblk):
    n = x.shape[0]
    grid = (n // nblk,)
    return pl.pallas_call(
        _prep_body,
        grid=grid,
        in_specs=[
            pl.BlockSpec((nblk, H), lambda i: (i, 0)),
            pl.BlockSpec((H, H), lambda i: (0, 0)),
            pl.BlockSpec((H, H), lambda i: (0, 0)),
            pl.BlockSpec((H, H), lambda i: (0, 0)),
            pl.BlockSpec((1, H), lambda i: (0, 0)),
        ],
        out_specs=[
            pl.BlockSpec((nblk, H), lambda i: (i, 0)),
            pl.BlockSpec((nblk, H), lambda i: (i, 0)),
            pl.BlockSpec((nblk, H), lambda i: (i, 0)),
        ],
        out_shape=[
            jax.ShapeDtypeStruct((n, H), jnp.int32),
            jax.ShapeDtypeStruct((n, H), F32),
            jax.ShapeDtypeStruct((n, H), F32),
        ],
    )(x, w1a, w1c, wd, bd)


def _sc_gather(tsrc, u3, src, dst):
    e = src.shape[0]
    per_w = e // SC_WORKERS
    n_full = per_w // 128
    tail = per_w - n_full * 128
    mesh = plsc.VectorSubcoreMesh(core_axis_name="c", subcore_axis_name="s")

    @functools.partial(
        pl.kernel,
        mesh=mesh,
        out_type=[
            jax.ShapeDtypeStruct((e, H), jnp.int32),
            jax.ShapeDtypeStruct((e, H), F32),
        ],
        scratch_types=[
            pltpu.VMEM((128,), jnp.int32),
            pltpu.VMEM((128, H), jnp.int32),
            pltpu.VMEM((128, H), F32),
            pltpu.SemaphoreType.DMA,
        ],
    )
    def k(tsrc_hbm, u3_hbm, src_hbm, dst_hbm, gs_hbm, gd_hbm, idxb, rs, rd, sem):
        c = lax.axis_index("c")
        s = lax.axis_index("s")
        wid = s * SC_CORES + c
        base = wid * per_w

        def chunk(off, n):
            pltpu.sync_copy(src_hbm.at[pl.ds(off, n)], idxb.at[pl.ds(0, n)])
            pltpu.async_copy(tsrc_hbm.at[idxb.at[pl.ds(0, n)]], rs.at[pl.ds(0, n)], sem).wait()
            pltpu.sync_copy(rs.at[pl.ds(0, n)], gs_hbm.at[pl.ds(off, n)])
            pltpu.sync_copy(dst_hbm.at[pl.ds(off, n)], idxb.at[pl.ds(0, n)])
            pltpu.async_copy(u3_hbm.at[idxb.at[pl.ds(0, n)]], rd.at[pl.ds(0, n)], sem).wait()
            pltpu.sync_copy(rd.at[pl.ds(0, n)], gd_hbm.at[pl.ds(off, n)])

        def body(j, carry):
            chunk(base + j * 128, 128)
            return carry

        lax.fori_loop(0, n_full, body, 0)
        if tail:
            chunk(base + n_full * 128, tail)

    return k(tsrc, u3, src, dst)


def _pass1_body(ea_ref, gs_ref, gd_ref, r_ref, cf_ref, wb_ref, bb_ref,
                wmid_ref, be1_ref, h1_ref, st_ref, acc):
    i = pl.program_id(0)
    a = jnp.dot(ea_ref[...], r_ref[...], preferred_element_type=F32)
    d = a - cf_ref[...]
    g = jnp.exp(-(d * d))
    g = _lrelu(jnp.dot(g, wb_ref[...], preferred_element_type=F32) + bb_ref[...])
    h1 = _lrelu(_unpack_lo(gs_ref[...]) + gd_ref[...] +
                jnp.dot(g, wmid_ref[...], preferred_element_type=F32) + be1_ref[...])
    h1_ref[...] = _pack_pair(h1[:, :H // 2], h1[:, H // 2:])

    @pl.when(i == 0)
    def _():
        acc[...] = jnp.zeros_like(acc)

    acc[0:1, :] += jnp.sum(h1, axis=0, keepdims=True)
    acc[1:2, :] += jnp.sum(h1 * h1, axis=0, keepdims=True)

    @pl.when(i == pl.num_programs(0) - 1)
    def _():
        st_ref[...] = acc[...]


def _pass1(ea, gs, gd, rmat, cf, wb, bb, wmid, be1, eb):
    e = ea.shape[0]
    grid = (e // eb,)
    return pl.pallas_call(
        _pass1_body,
        grid=grid,
        in_specs=[
            pl.BlockSpec((eb, 4), lambda i: (i, 0)),
            pl.BlockSpec((eb, H), lambda i: (i, 0)),
            pl.BlockSpec((eb, H), lambda i: (i, 0)),
            pl.BlockSpec((4, 4 * NSTEP), lambda i: (0, 0)),
            pl.BlockSpec((1, 4 * NSTEP), lambda i: (0, 0)),
            pl.BlockSpec((4 * NSTEP, H), lambda i: (0, 0)),
            pl.BlockSpec((1, H), lambda i: (0, 0)),
            pl.BlockSpec((H, H), lambda i: (0, 0)),
            pl.BlockSpec((1, H), lambda i: (0, 0)),
        ],
        out_specs=[
            pl.BlockSpec((eb, H // 2), lambda i: (i, 0)),
            pl.BlockSpec((8, H), lambda i: (0, 0)),
        ],
        out_shape=[
            jax.ShapeDtypeStruct((e, H // 2), jnp.int32),
            jax.ShapeDtypeStruct((8, H), F32),
        ],
        scratch_shapes=[pltpu.VMEM((8, H), F32)],
    )(ea, gs, gd, rmat, cf, wb, bb, wmid, be1)


def _bn_vecs(st_ref, ne, g_ref, bt_ref):
    m = st_ref[0:1, :] * (1.0 / ne)
    var = st_ref[1:2, :] * (1.0 / ne) - m * m
    sc = g_ref[...] * lax.rsqrt(var + EPS)
    return m, sc, bt_ref[...]


def _stats2_body(h1_ref, st1_ref, ge1_ref, bte1_ref, we2_ref, be2_ref,
                 st2_ref, acc, *, ne):
    i = pl.program_id(0)
    m1, sc1, bt1 = _bn_vecs(st1_ref, ne, ge1_ref, bte1_ref)
    h1p = h1_ref[...]
    h1 = jnp.concatenate([_unpack_lo(h1p), _unpack_hi(h1p)], axis=1)
    hn = (h1 - m1) * sc1 + bt1
    h2 = _lrelu(jnp.dot(hn, we2_ref[...], preferred_element_type=F32) + be2_ref[...])

    @pl.when(i == 0)
    def _():
        acc[...] = jnp.zeros_like(acc)

    acc[0:1, :] += jnp.sum(h2, axis=0, keepdims=True)
    acc[1:2, :] += jnp.sum(h2 * h2, axis=0, keepdims=True)

    @pl.when(i == pl.num_programs(0) - 1)
    def _():
        st2_ref[...] = acc[...]


def _stats2(h1, st1, ge1, bte1, we2, be2, eb):
    e = h1.shape[0]
    grid = (e // eb,)
    return pl.pallas_call(
        functools.partial(_stats2_body, ne=e),
        grid=grid,
        in_specs=[
            pl.BlockSpec((eb, H // 2), lambda i: (i, 0)),
            pl.BlockSpec((8, H), lambda i: (0, 0)),
            pl.BlockSpec((1, H), lambda i: (0, 0)),
            pl.BlockSpec((1, H), lambda i: (0, 0)),
            pl.BlockSpec((H, H), lambda i: (0, 0)),
            pl.BlockSpec((1, H), lambda i: (0, 0)),
        ],
        out_specs=pl.BlockSpec((8, H), lambda i: (0, 0)),
        out_shape=jax.ShapeDtypeStruct((8, H), F32),
        scratch_shapes=[pltpu.VMEM((8, H), F32)],
    )(h1, st1, ge1, bte1, we2, be2)


def _msgs_body(h1_ref, gxd_ref, ea_ref, st1_ref, ge1_ref, bte1_ref,
               we2_ref, be2_ref, st2_ref, ge2_ref, bte2_ref,
               we3_ref, be3_ref, m_ref, *, ne):
    m1, sc1, bt1 = _bn_vecs(st1_ref, ne, ge1_ref, bte1_ref)
    h1p = h1_ref[...]
    h1 = jnp.concatenate([_unpack_lo(h1p), _unpack_hi(h1p)], axis=1)
    hn = (h1 - m1) * sc1 + bt1
    h2 = _lrelu(jnp.dot(hn, we2_ref[...], preferred_element_type=F32) + be2_ref[...])
    m2, sc2, bt2 = _bn_vecs(st2_ref, ne, ge2_ref, bte2_ref)
    hn2 = (h2 - m2) * sc2 + bt2
    h3 = jnp.dot(hn2, we3_ref[...], preferred_element_type=F32) + be3_ref[...]
    coef = jnp.cos((np.pi / 2) * ea_ref[:, 3:4])
    m_ref[...] = coef * h3 * _unpack_hi(gxd_ref[...])


def _msgs(h1, gs, ea, st1, ge1, bte1, we2, be2, st2, ge2, bte2, we3, be3, eb):
    e = h1.shape[0]
    grid = (e // eb,)
    return pl.pallas_call(
        functools.partial(_msgs_body, ne=e),
        grid=grid,
        in_specs=[
            pl.BlockSpec((eb, H // 2), lambda i: (i, 0)),
            pl.BlockSpec((eb, H), lambda i: (i, 0)),
            pl.BlockSpec((eb, 4), lambda i: (i, 0)),
            pl.BlockSpec((8, H), lambda i: (0, 0)),
            pl.BlockSpec((1, H), lambda i: (0, 0)),
            pl.BlockSpec((1, H), lambda i: (0, 0)),
            pl.BlockSpec((H, H), lambda i: (0, 0)),
            pl.BlockSpec((1, H), lambda i: (0, 0)),
            pl.BlockSpec((8, H), lambda i: (0, 0)),
            pl.BlockSpec((1, H), lambda i: (0, 0)),
            pl.BlockSpec((1, H), lambda i: (0, 0)),
            pl.BlockSpec((H, H), lambda i: (0, 0)),
            pl.BlockSpec((1, H), lambda i: (0, 0)),
        ],
        out_specs=pl.BlockSpec((eb, H), lambda i: (i, 0)),
        out_shape=jax.ShapeDtypeStruct((e, H), F32),
    )(h1, gs, ea, st1, ge1, bte1, we2, be2, st2, ge2, bte2, we3, be3)


def _sc_scatter(m, dst, zeros):
    e = m.shape[0]
    n = zeros.shape[0]
    per_w = e // SC_WORKERS
    n_full = per_w // 128
    tail = per_w - n_full * 128
    mesh = plsc.VectorSubcoreMesh(core_axis_name="c", subcore_axis_name="s")

    @functools.partial(
        pl.kernel,
        mesh=mesh,
        out_type=jax.ShapeDtypeStruct((SC_CORES, n, H), F32),
        scratch_types=[
            pltpu.VMEM((128,), jnp.int32),
            pltpu.VMEM((128, H), F32),
            pltpu.VMEM_SHARED((n, H), F32),
        ],
    )
    def k(m_hbm, dst_hbm, z_hbm, out_hbm, idxb, rb, accsh):
        c = lax.axis_index("c")
        s = lax.axis_index("s")
        r0 = s * 640
        ncp = jnp.where(s == SC_SUBCORES - 1, (n - 640 * (SC_SUBCORES - 1)) // 40,
                        640 // 40)

        def cp_init(j, carry):
            off = r0 + j * 40
            pltpu.sync_copy(z_hbm.at[pl.ds(off, 40)], accsh.at[pl.ds(off, 40)])
            return carry

        lax.fori_loop(0, ncp, cp_init, 0)
        plsc.subcore_barrier()
        base = (c * SC_SUBCORES + s) * per_w

        def chunk(off, nn):
            pltpu.sync_copy(dst_hbm.at[pl.ds(off, nn)], idxb.at[pl.ds(0, nn)])
            pltpu.sync_copy(m_hbm.at[pl.ds(off, nn)], rb.at[pl.ds(0, nn)])
            pltpu.sync_copy(rb.at[pl.ds(0, nn)], accsh.at[idxb.at[pl.ds(0, nn)]], add=True)

        def body(j, carry):
            chunk(base + j * 128, 128)
            return carry

        lax.fori_loop(0, n_full, body, 0)
        if tail:
            chunk(base + n_full * 128, tail)
        plsc.subcore_barrier()

        def cp_out(j, carry):
            off = r0 + j * 40
            pltpu.sync_copy(accsh.at[pl.ds(off, 40)], out_hbm.at[c, pl.ds(off, 40)])
            return carry

        lax.fori_loop(0, ncp, cp_out, 0)

    return k(m, dst, zeros)


def _node_stats_body(xd_ref, i0_ref, i1_ref, v_ref, wn1_ref, bn1_ref,
                     st_ref, acc):
    i = pl.program_id(0)
    z0 = v_ref[...] * xd_ref[...] + i0_ref[0] + i1_ref[0]
    n1 = _lrelu(jnp.dot(z0, wn1_ref[...], preferred_element_type=F32) + bn1_ref[...])

    @pl.when(i == 0)
    def _():
        acc[...] = jnp.zeros_like(acc)

    acc[0:1, :] += jnp.sum(n1, axis=0, keepdims=True)
    acc[1:2, :] += jnp.sum(n1 * n1, axis=0, keepdims=True)

    @pl.when(i == pl.num_programs(0) - 1)
    def _():
        st_ref[...] = acc[...]


def _node_apply_body(xd_ref, i0_ref, i1_ref, x_ref, st_ref, v_ref, wn1_ref,
                     bn1_ref, gn1_ref, btn1_ref, wn2_ref, bn2_ref, z_ref, *, nn):
    z0 = v_ref[...] * xd_ref[...] + i0_ref[0] + i1_ref[0]
    n1 = _lrelu(jnp.dot(z0, wn1_ref[...], preferred_element_type=F32) + bn1_ref[...])
    m, sc, bt = _bn_vecs(st_ref, nn, gn1_ref, btn1_ref)
    nbn = (n1 - m) * sc + bt
    z_ref[...] = (jnp.dot(nbn, wn2_ref[...], preferred_element_type=F32)
                  + bn2_ref[...] + x_ref[...])


def _node(xd, inc, x, v, wn1, bn1, gn1, btn1, wn2, bn2, nblk):
    n = x.shape[0]
    grid = (n // nblk,)
    blk = lambda shape: pl.BlockSpec(shape, lambda i: (0,) * len(shape))
    nodeb = pl.BlockSpec((nblk, H), lambda i: (i, 0))
    inc0 = pl.BlockSpec((1, nblk, H), lambda i: (0, i, 0))
    inc1 = pl.BlockSpec((1, nblk, H), lambda i: (1, i, 0))
    st = pl.pallas_call(
        _node_stats_body,
        grid=grid,
        in_specs=[nodeb, inc0, inc1, blk((1, H)), blk((H, H)), blk((1, H))],
        out_specs=pl.BlockSpec((8, H), lambda i: (0, 0)),
        out_shape=jax.ShapeDtypeStruct((8, H), F32),
        scratch_shapes=[pltpu.VMEM((8, H), F32)],
    )(xd, inc, inc, v, wn1, bn1)
    return pl.pallas_call(
        functools.partial(_node_apply_body, nn=n),
        grid=grid,
        in_specs=[nodeb, inc0, inc1, nodeb, blk((8, H)), blk((1, H)),
                  blk((H, H)), blk((1, H)), blk((1, H)), blk((1, H)),
                  blk((H, H)), blk((1, H))],
        out_specs=nodeb,
        out_shape=jax.ShapeDtypeStruct((n, H), F32),
    )(xd, inc, inc, x, st, v, wn1, bn1, gn1, btn1, wn2, bn2)


def kernel(x, edge_attr, edge_index, Wb, bb, w_e1, b_e1, g_e1, bt_e1,
           w_e2, b_e2, g_e2, bt_e2, w_e3, b_e3, w_d, b_d, v,
           w_n1, b_n1, g_n1, bt_n1, w_n2, b_n2):
    n = x.shape[0]
    e = edge_attr.shape[0]
    src = edge_index[0]
    dst = edge_index[1]

    w1a = w_e1[0:H]
    wmid = w_e1[H:2 * H]
    w1c = w_e1[2 * H:3 * H]

    row = lambda b: b.reshape(1, -1)

    rmat = jnp.repeat(jnp.eye(4, dtype=F32) * NSTEP, NSTEP, axis=1)
    cf = (jnp.tile(jnp.linspace(0.0, 1.0, NSTEP), 4) * NSTEP).astype(F32).reshape(1, -1)

    tsrc, u3, xd = _prep(x, w1a, w1c, w_d, row(b_d), nblk=2000)
    gs, gd = _sc_gather(tsrc, u3, src, dst)

    eb = 2560
    h1, st1 = _pass1(edge_attr, gs, gd, rmat, cf, Wb, row(bb), wmid,
                     row(b_e1), eb)
    st2 = _stats2(h1, st1, row(g_e1), row(bt_e1), w_e2, row(b_e2), eb)
    m = _msgs(h1, gs, edge_attr, st1, row(g_e1), row(bt_e1), w_e2, row(b_e2),
              st2, row(g_e2), row(bt_e2), w_e3, row(b_e3), eb)

    zeros = jnp.zeros((n, H), F32)
    inc = _sc_scatter(m, dst, zeros)

    return _node(xd, inc, x, v, w_n1, row(b_n1), g_n1.reshape(1, -1),
                 bt_n1.reshape(1, -1), w_n2, row(b_n2), nblk=2000)


# speedup vs baseline: 1.1363x; 1.1363x over previous
import functools

import numpy as np
import jax
import jax.numpy as jnp
from jax import lax
from jax.experimental import pallas as pl
from jax.experimental.pallas import tpu as pltpu
from jax.experimental.pallas import tpu_sc as plsc

F32 = jnp.float32
BF16 = jnp.bfloat16
H = 128
NSTEP = 50
EPS = 1e-5

SC_CORES = 2
SC_SUBCORES = 16
SC_WORKERS = SC_CORES * SC_SUBCORES


def _lrelu(x):
    return jnp.where(x >= 0, x, 0.01 * x)


_MASK_HI = -65536
_MASK_LO = 0xFFFF


def _rne_bf16_bits(f):
    b = lax.bitcast_convert_type(f, jnp.int32)
    return b + 0x7FFF + ((b >> 16) & 1)


def _pack_pair(lo, hi):
    return (((_rne_bf16_bits(lo) >> 16) & _MASK_LO)
            | (_rne_bf16_bits(hi) & _MASK_HI))


def _unpack_lo(p):
    return lax.bitcast_convert_type(p << 16, F32)


def _unpack_hi(p):
    return lax.bitcast_convert_type(p & _MASK_HI, F32)


def _prep_body(x_ref, w1a_ref, w1c_ref, wd_ref, bd_ref, tsrc_ref, u3_ref,
               xd_ref):
    xb = x_ref[...]
    xd = jnp.dot(xb, wd_ref[...], preferred_element_type=F32) + bd_ref[...]
    u1 = jnp.dot(xb, w1a_ref[...], preferred_element_type=F32)
    tsrc_ref[...] = _pack_pair(u1, xd)
    u3_ref[...] = jnp.dot(xb, w1c_ref[...], preferred_element_type=F32)
    xd_ref[...] = xd


def _prep(x, w1a, w1c, wd, bd, nblk):
    n = x.shape[0]
    grid = (n // nblk,)
    return pl.pallas_call(
        _prep_body,
        grid=grid,
        in_specs=[
            pl.BlockSpec((nblk, H), lambda i: (i, 0)),
            pl.BlockSpec((H, H), lambda i: (0, 0)),
            pl.BlockSpec((H, H), lambda i: (0, 0)),
            pl.BlockSpec((H, H), lambda i: (0, 0)),
            pl.BlockSpec((1, H), lambda i: (0, 0)),
        ],
        out_specs=[
            pl.BlockSpec((nblk, H), lambda i: (i, 0)),
            pl.BlockSpec((nblk, H), lambda i: (i, 0)),
            pl.BlockSpec((nblk, H), lambda i: (i, 0)),
        ],
        out_shape=[
            jax.ShapeDtypeStruct((n, H), jnp.int32),
            jax.ShapeDtypeStruct((n, H), F32),
            jax.ShapeDtypeStruct((n, H), F32),
        ],
    )(x, w1a, w1c, wd, bd)


def _sc_gather(tsrc, u3, src, dst):
    e = src.shape[0]
    per_w = e // SC_WORKERS
    n_full = per_w // 128
    tail = per_w - n_full * 128
    mesh = plsc.VectorSubcoreMesh(core_axis_name="c", subcore_axis_name="s")

    @functools.partial(
        pl.kernel,
        mesh=mesh,
        out_type=[
            jax.ShapeDtypeStruct((e, H), jnp.int32),
            jax.ShapeDtypeStruct((e, H), F32),
        ],
        scratch_types=[
            pltpu.VMEM((per_w,), jnp.int32),
            pltpu.VMEM((per_w,), jnp.int32),
            pltpu.VMEM((128, H), jnp.int32),
            pltpu.VMEM((128, H), jnp.int32),
            pltpu.VMEM((128, H), F32),
            pltpu.VMEM((128, H), F32),
            pltpu.SemaphoreType.DMA,
            pltpu.SemaphoreType.DMA,
            pltpu.SemaphoreType.DMA,
            pltpu.SemaphoreType.DMA,
        ],
    )
    def k(tsrc_hbm, u3_hbm, src_hbm, dst_hbm, gs_hbm, gd_hbm,
          idxs, idxd, rs0, rs1, rd0, rd1, sem_s0, sem_s1, sem_d0, sem_d1):
        c = lax.axis_index("c")
        s = lax.axis_index("s")
        wid = s * SC_CORES + c
        base = wid * per_w

        pltpu.sync_copy(src_hbm.at[pl.ds(base, per_w)], idxs)
        pltpu.sync_copy(dst_hbm.at[pl.ds(base, per_w)], idxd)

        def start(j, rbuf_s, rbuf_d, sem_a, sem_b):
            pltpu.async_copy(tsrc_hbm.at[idxs.at[pl.ds(j * 128, 128)]], rbuf_s, sem_a)
            pltpu.async_copy(u3_hbm.at[idxd.at[pl.ds(j * 128, 128)]], rbuf_d, sem_b)

        def finish(j, rbuf_s, rbuf_d, sem_a, sem_b):
            pltpu.make_async_copy(tsrc_hbm.at[idxs.at[pl.ds(j * 128, 128)]], rbuf_s, sem_a).wait()
            pltpu.make_async_copy(u3_hbm.at[idxd.at[pl.ds(j * 128, 128)]], rbuf_d, sem_b).wait()
            pltpu.sync_copy(rbuf_s, gs_hbm.at[pl.ds(base + j * 128, 128)])
            pltpu.sync_copy(rbuf_d, gd_hbm.at[pl.ds(base + j * 128, 128)])

        start(0, rs0, rd0, sem_s0, sem_d0)

        def body(j2, carry):
            j = j2 * 2
            start(j + 1, rs1, rd1, sem_s1, sem_d1)
            finish(j, rs0, rd0, sem_s0, sem_d0)

            @pl.when(j + 2 < n_full)
            def _():
                start(j + 2, rs0, rd0, sem_s0, sem_d0)

            finish(j + 1, rs1, rd1, sem_s1, sem_d1)
            return carry

        lax.fori_loop(0, n_full // 2, body, 0)
        if tail:
            off = base + n_full * 128
            pltpu.sync_copy(src_hbm.at[pl.ds(off, tail)], idxs.at[pl.ds(0, tail)])
            pltpu.async_copy(tsrc_hbm.at[idxs.at[pl.ds(0, tail)]],
                             rs0.at[pl.ds(0, tail)], sem_s0).wait()
            pltpu.sync_copy(rs0.at[pl.ds(0, tail)], gs_hbm.at[pl.ds(off, tail)])
            pltpu.sync_copy(dst_hbm.at[pl.ds(off, tail)], idxd.at[pl.ds(0, tail)])
            pltpu.async_copy(u3_hbm.at[idxd.at[pl.ds(0, tail)]],
                             rd0.at[pl.ds(0, tail)], sem_d0).wait()
            pltpu.sync_copy(rd0.at[pl.ds(0, tail)], gd_hbm.at[pl.ds(off, tail)])

    return k(tsrc, u3, src, dst)


def _pass1_body(ea_ref, gs_ref, gd_ref, r_ref, cf_ref, wb_ref, bb_ref,
                wmid_ref, be1_ref, h1_ref, st_ref, acc):
    i = pl.program_id(0)
    a = jnp.dot(ea_ref[...], r_ref[...], preferred_element_type=F32)
    d = a - cf_ref[...]
    g = jnp.exp(-(d * d))
    g = _lrelu(jnp.dot(g, wb_ref[...], preferred_element_type=F32) + bb_ref[...])
    h1 = _lrelu(_unpack_lo(gs_ref[...]) + gd_ref[...] +
                jnp.dot(g, wmid_ref[...], preferred_element_type=F32) + be1_ref[...])
    h1_ref[...] = h1

    @pl.when(i == 0)
    def _():
        acc[...] = jnp.zeros_like(acc)

    acc[0:1, :] += jnp.sum(h1, axis=0, keepdims=True)
    acc[1:2, :] += jnp.sum(h1 * h1, axis=0, keepdims=True)

    @pl.when(i == pl.num_programs(0) - 1)
    def _():
        st_ref[...] = acc[...]


def _pass1(ea, gs, gd, rmat, cf, wb, bb, wmid, be1, eb):
    e = ea.shape[0]
    grid = (e // eb,)
    return pl.pallas_call(
        _pass1_body,
        grid=grid,
        in_specs=[
            pl.BlockSpec((eb, 4), lambda i: (i, 0)),
            pl.BlockSpec((eb, H), lambda i: (i, 0)),
            pl.BlockSpec((eb, H), lambda i: (i, 0)),
            pl.BlockSpec((4, 4 * NSTEP), lambda i: (0, 0)),
            pl.BlockSpec((1, 4 * NSTEP), lambda i: (0, 0)),
            pl.BlockSpec((4 * NSTEP, H), lambda i: (0, 0)),
            pl.BlockSpec((1, H), lambda i: (0, 0)),
            pl.BlockSpec((H, H), lambda i: (0, 0)),
            pl.BlockSpec((1, H), lambda i: (0, 0)),
        ],
        out_specs=[
            pl.BlockSpec((eb, H), lambda i: (i, 0)),
            pl.BlockSpec((8, H), lambda i: (0, 0)),
        ],
        out_shape=[
            jax.ShapeDtypeStruct((e, H), F32),
            jax.ShapeDtypeStruct((8, H), F32),
        ],
        scratch_shapes=[pltpu.VMEM((8, H), F32)],
    )(ea, gs, gd, rmat, cf, wb, bb, wmid, be1)


def _bn_vecs(st_ref, ne, g_ref, bt_ref):
    m = st_ref[0:1, :] * (1.0 / ne)
    var = st_ref[1:2, :] * (1.0 / ne) - m * m
    sc = g_ref[...] * lax.rsqrt(var + EPS)
    return m, sc, bt_ref[...]


def _stats2_body(h1_ref, st1_ref, ge1_ref, bte1_ref, we2_ref, be2_ref,
                 st2_ref, acc, *, ne):
    i = pl.program_id(0)
    m1, sc1, bt1 = _bn_vecs(st1_ref, ne, ge1_ref, bte1_ref)
    hn = (h1_ref[...] - m1) * sc1 + bt1
    h2 = _lrelu(jnp.dot(hn, we2_ref[...], preferred_element_type=F32) + be2_ref[...])

    @pl.when(i == 0)
    def _():
        acc[...] = jnp.zeros_like(acc)

    acc[0:1, :] += jnp.sum(h2, axis=0, keepdims=True)
    acc[1:2, :] += jnp.sum(h2 * h2, axis=0, keepdims=True)

    @pl.when(i == pl.num_programs(0) - 1)
    def _():
        st2_ref[...] = acc[...]


def _stats2(h1, st1, ge1, bte1, we2, be2, eb):
    e = h1.shape[0]
    grid = (e // eb,)
    return pl.pallas_call(
        functools.partial(_stats2_body, ne=e),
        grid=grid,
        in_specs=[
            pl.BlockSpec((eb, H), lambda i: (i, 0)),
            pl.BlockSpec((8, H), lambda i: (0, 0)),
            pl.BlockSpec((1, H), lambda i: (0, 0)),
            pl.BlockSpec((1, H), lambda i: (0, 0)),
            pl.BlockSpec((H, H), lambda i: (0, 0)),
            pl.BlockSpec((1, H), lambda i: (0, 0)),
        ],
        out_specs=pl.BlockSpec((8, H), lambda i: (0, 0)),
        out_shape=jax.ShapeDtypeStruct((8, H), F32),
        scratch_shapes=[pltpu.VMEM((8, H), F32)],
    )(h1, st1, ge1, bte1, we2, be2)


def _msgs_body(h1_ref, gxd_ref, ea_ref, st1_ref, ge1_ref, bte1_ref,
               we2_ref, be2_ref, st2_ref, ge2_ref, bte2_ref,
               we3_ref, be3_ref, m_ref, *, ne):
    m1, sc1, bt1 = _bn_vecs(st1_ref, ne, ge1_ref, bte1_ref)
    hn = (h1_ref[...] - m1) * sc1 + bt1
    h2 = _lrelu(jnp.dot(hn, we2_ref[...], preferred_element_type=F32) + be2_ref[...])
    m2, sc2, bt2 = _bn_vecs(st2_ref, ne, ge2_ref, bte2_ref)
    hn2 = (h2 - m2) * sc2 + bt2
    h3 = jnp.dot(hn2, we3_ref[...], preferred_element_type=F32) + be3_ref[...]
    coef = jnp.cos((np.pi / 2) * ea_ref[:, 3:4])
    m_ref[...] = coef * h3 * _unpack_hi(gxd_ref[...])


def _msgs(h1, gs, ea, st1, ge1, bte1, we2, be2, st2, ge2, bte2, we3, be3, eb):
    e = h1.shape[0]
    grid = (e // eb,)
    return pl.pallas_call(
        functools.partial(_msgs_body, ne=e),
        grid=grid,
        in_specs=[
            pl.BlockSpec((eb, H), lambda i: (i, 0)),
            pl.BlockSpec((eb, H), lambda i: (i, 0)),
            pl.BlockSpec((eb, 4), lambda i: (i, 0)),
            pl.BlockSpec((8, H), lambda i: (0, 0)),
            pl.BlockSpec((1, H), lambda i: (0, 0)),
            pl.BlockSpec((1, H), lambda i: (0, 0)),
            pl.BlockSpec((H, H), lambda i: (0, 0)),
            pl.BlockSpec((1, H), lambda i: (0, 0)),
            pl.BlockSpec((8, H), lambda i: (0, 0)),
            pl.BlockSpec((1, H), lambda i: (0, 0)),
            pl.BlockSpec((1, H), lambda i: (0, 0)),
            pl.BlockSpec((H, H), lambda i: (0, 0)),
            pl.BlockSpec((1, H), lambda i: (0, 0)),
        ],
        out_specs=pl.BlockSpec((eb, H), lambda i: (i, 0)),
        out_shape=jax.ShapeDtypeStruct((e, H), F32),
    )(h1, gs, ea, st1, ge1, bte1, we2, be2, st2, ge2, bte2, we3, be3)


def _sc_scatter(m, dst, zeros):
    e = m.shape[0]
    n = zeros.shape[0]
    per_w = e // SC_WORKERS
    n_full = per_w // 128
    tail = per_w - n_full * 128
    mesh = plsc.VectorSubcoreMesh(core_axis_name="c", subcore_axis_name="s")

    @functools.partial(
        pl.kernel,
        mesh=mesh,
        out_type=jax.ShapeDtypeStruct((SC_CORES, n, H), F32),
        scratch_types=[
            pltpu.VMEM((128,), jnp.int32),
            pltpu.VMEM((128, H), F32),
            pltpu.VMEM_SHARED((n, H), F32),
        ],
    )
    def k(m_hbm, dst_hbm, z_hbm, out_hbm, idxb, rb, accsh):
        c = lax.axis_index("c")
        s = lax.axis_index("s")
        r0 = s * 640
        ncp = jnp.where(s == SC_SUBCORES - 1, (n - 640 * (SC_SUBCORES - 1)) // 40,
                        640 // 40)

        def cp_init(j, carry):
            off = r0 + j * 40
            pltpu.sync_copy(z_hbm.at[pl.ds(off, 40)], accsh.at[pl.ds(off, 40)])
            return carry

        lax.fori_loop(0, ncp, cp_init, 0)
        plsc.subcore_barrier()
        base = (c * SC_SUBCORES + s) * per_w

        def chunk(off, nn):
            pltpu.sync_copy(dst_hbm.at[pl.ds(off, nn)], idxb.at[pl.ds(0, nn)])
            pltpu.sync_copy(m_hbm.at[pl.ds(off, nn)], rb.at[pl.ds(0, nn)])
            pltpu.sync_copy(rb.at[pl.ds(0, nn)], accsh.at[idxb.at[pl.ds(0, nn)]], add=True)

        def body(j, carry):
            chunk(base + j * 128, 128)
            return carry

        lax.fori_loop(0, n_full, body, 0)
        if tail:
            chunk(base + n_full * 128, tail)
        plsc.subcore_barrier()

        def cp_out(j, carry):
            off = r0 + j * 40
            pltpu.sync_copy(accsh.at[pl.ds(off, 40)], out_hbm.at[c, pl.ds(off, 40)])
            return carry

        lax.fori_loop(0, ncp, cp_out, 0)

    return k(m, dst, zeros)


def _node_stats_body(xd_ref, i0_ref, i1_ref, v_ref, wn1_ref, bn1_ref,
                     st_ref, acc):
    i = pl.program_id(0)
    z0 = v_ref[...] * xd_ref[...] + i0_ref[0] + i1_ref[0]
    n1 = _lrelu(jnp.dot(z0, wn1_ref[...], preferred_element_type=F32) + bn1_ref[...])

    @pl.when(i == 0)
    def _():
        acc[...] = jnp.zeros_like(acc)

    acc[0:1, :] += jnp.sum(n1, axis=0, keepdims=True)
    acc[1:2, :] += jnp.sum(n1 * n1, axis=0, keepdims=True)

    @pl.when(i == pl.num_programs(0) - 1)
    def _():
        st_ref[...] = acc[...]


def _node_apply_body(xd_ref, i0_ref, i1_ref, x_ref, st_ref, v_ref, wn1_ref,
                     bn1_ref, gn1_ref, btn1_ref, wn2_ref, bn2_ref, z_ref, *, nn):
    z0 = v_ref[...] * xd_ref[...] + i0_ref[0] + i1_ref[0]
    n1 = _lrelu(jnp.dot(z0, wn1_ref[...], preferred_element_type=F32) + bn1_ref[...])
    m, sc, bt = _bn_vecs(st_ref, nn, gn1_ref, btn1_ref)
    nbn = (n1 - m) * sc + bt
    z_ref[...] = (jnp.dot(nbn, wn2_ref[...], preferred_element_type=F32)
                  + bn2_ref[...] + x_ref[...])


def _node(xd, inc, x, v, wn1, bn1, gn1, btn1, wn2, bn2, nblk):
    n = x.shape[0]
    grid = (n // nblk,)
    blk = lambda shape: pl.BlockSpec(shape, lambda i: (0,) * len(shape))
    nodeb = pl.BlockSpec((nblk, H), lambda i: (i, 0))
    inc0 = pl.BlockSpec((1, nblk, H), lambda i: (0, i, 0))
    inc1 = pl.BlockSpec((1, nblk, H), lambda i: (1, i, 0))
    st = pl.pallas_call(
        _node_stats_body,
        grid=grid,
        in_specs=[nodeb, inc0, inc1, blk((1, H)), blk((H, H)), blk((1, H))],
        out_specs=pl.BlockSpec((8, H), lambda i: (0, 0)),
        out_shape=jax.ShapeDtypeStruct((8, H), F32),
        scratch_shapes=[pltpu.VMEM((8, H), F32)],
    )(xd, inc, inc, v, wn1, bn1)
    return pl.pallas_call(
        functools.partial(_node_apply_body, nn=n),
        grid=grid,
        in_specs=[nodeb, inc0, inc1, nodeb, blk((8, H)), blk((1, H)),
                  blk((H, H)), blk((1, H)), blk((1, H)), blk((1, H)),
                  blk((H, H)), blk((1, H))],
        out_specs=nodeb,
        out_shape=jax.ShapeDtypeStruct((n, H), F32),
    )(xd, inc, inc, x, st, v, wn1, bn1, gn1, btn1, wn2, bn2)


def kernel(x, edge_attr, edge_index, Wb, bb, w_e1, b_e1, g_e1, bt_e1,
           w_e2, b_e2, g_e2, bt_e2, w_e3, b_e3, w_d, b_d, v,
           w_n1, b_n1, g_n1, bt_n1, w_n2, b_n2):
    n = x.shape[0]
    e = edge_attr.shape[0]
    src = edge_index[0]
    dst = edge_index[1]

    w1a = w_e1[0:H]
    wmid = w_e1[H:2 * H]
    w1c = w_e1[2 * H:3 * H]

    row = lambda b: b.reshape(1, -1)

    rmat = jnp.repeat(jnp.eye(4, dtype=F32) * NSTEP, NSTEP, axis=1)
    cf = (jnp.tile(jnp.linspace(0.0, 1.0, NSTEP), 4) * NSTEP).astype(F32).reshape(1, -1)

    tsrc, u3, xd = _prep(x, w1a, w1c, w_d, row(b_d), nblk=2000)
    gs, gd = _sc_gather(tsrc, u3, src, dst)

    eb = 2560
    h1, st1 = _pass1(edge_attr, gs, gd, rmat, cf, Wb, row(bb), wmid,
                     row(b_e1), eb)
    st2 = _stats2(h1, st1, row(g_e1), row(bt_e1), w_e2, row(b_e2), eb)
    m = _msgs(h1, gs, edge_attr, st1, row(g_e1), row(bt_e1), w_e2, row(b_e2),
              st2, row(g_e2), row(bt_e2), w_e3, row(b_e3), eb)

    zeros = jnp.zeros((n, H), F32)
    inc = _sc_scatter(m, dst, zeros)

    return _node(xd, inc, x, v, w_n1, row(b_n1), g_n1.reshape(1, -1),
                 bt_n1.reshape(1, -1), w_n2, row(b_n2), nblk=2000)


# speedup vs baseline: 1.2155x; 1.0697x over previous
import functools

import numpy as np
import jax
import jax.numpy as jnp
from jax import lax
from jax.experimental import pallas as pl
from jax.experimental.pallas import tpu as pltpu
from jax.experimental.pallas import tpu_sc as plsc

F32 = jnp.float32
BF16 = jnp.bfloat16
H = 128
NSTEP = 50
EPS = 1e-5

SC_CORES = 2
SC_SUBCORES = 16
SC_WORKERS = SC_CORES * SC_SUBCORES


def _lrelu(x):
    return jnp.where(x >= 0, x, 0.01 * x)


_MASK_HI = -65536
_MASK_LO = 0xFFFF


def _rne_bf16_bits(f):
    b = lax.bitcast_convert_type(f, jnp.int32)
    return b + 0x7FFF + ((b >> 16) & 1)


def _pack_pair(lo, hi):
    return (((_rne_bf16_bits(lo) >> 16) & _MASK_LO)
            | (_rne_bf16_bits(hi) & _MASK_HI))


def _unpack_lo(p):
    return lax.bitcast_convert_type(p << 16, F32)


def _unpack_hi(p):
    return lax.bitcast_convert_type(p & _MASK_HI, F32)


def _prep_body(x_ref, w1a_ref, w1c_ref, wd_ref, bd_ref, tsrc_ref, u3_ref,
               xd_ref):
    xb = x_ref[...]
    xd = jnp.dot(xb, wd_ref[...], preferred_element_type=F32) + bd_ref[...]
    u1 = jnp.dot(xb, w1a_ref[...], preferred_element_type=F32)
    tsrc_ref[...] = _pack_pair(u1, xd)
    u3_ref[...] = jnp.dot(xb, w1c_ref[...], preferred_element_type=F32)
    xd_ref[...] = xd


def _prep(x, w1a, w1c, wd, bd, nblk):
    n = x.shape[0]
    grid = (n // nblk,)
    return pl.pallas_call(
        _prep_body,
        grid=grid,
        in_specs=[
            pl.BlockSpec((nblk, H), lambda i: (i, 0)),
            pl.BlockSpec((H, H), lambda i: (0, 0)),
            pl.BlockSpec((H, H), lambda i: (0, 0)),
            pl.BlockSpec((H, H), lambda i: (0, 0)),
            pl.BlockSpec((1, H), lambda i: (0, 0)),
        ],
        out_specs=[
            pl.BlockSpec((nblk, H), lambda i: (i, 0)),
            pl.BlockSpec((nblk, H), lambda i: (i, 0)),
            pl.BlockSpec((nblk, H), lambda i: (i, 0)),
        ],
        out_shape=[
            jax.ShapeDtypeStruct((n, H), jnp.int32),
            jax.ShapeDtypeStruct((n, H), F32),
            jax.ShapeDtypeStruct((n, H), F32),
        ],
    )(x, w1a, w1c, wd, bd)


def _sc_gather(tsrc, u3, src, dst):
    e = src.shape[0]
    per_w = e // SC_WORKERS
    n_full = per_w // 128
    tail = per_w - n_full * 128
    mesh = plsc.VectorSubcoreMesh(core_axis_name="c", subcore_axis_name="s")

    @functools.partial(
        pl.kernel,
        mesh=mesh,
        out_type=[
            jax.ShapeDtypeStruct((e, H), jnp.int32),
            jax.ShapeDtypeStruct((e, H), F32),
        ],
        scratch_types=[
            pltpu.VMEM((per_w,), jnp.int32),
            pltpu.VMEM((per_w,), jnp.int32),
            pltpu.VMEM((128, H), jnp.int32),
            pltpu.VMEM((128, H), jnp.int32),
            pltpu.VMEM((128, H), F32),
            pltpu.VMEM((128, H), F32),
            pltpu.SemaphoreType.DMA,
            pltpu.SemaphoreType.DMA,
            pltpu.SemaphoreType.DMA,
            pltpu.SemaphoreType.DMA,
        ],
    )
    def k(tsrc_hbm, u3_hbm, src_hbm, dst_hbm, gs_hbm, gd_hbm,
          idxs, idxd, rs0, rs1, rd0, rd1, sem_s0, sem_s1, sem_d0, sem_d1):
        c = lax.axis_index("c")
        s = lax.axis_index("s")
        wid = s * SC_CORES + c
        base = wid * per_w

        pltpu.sync_copy(src_hbm.at[pl.ds(base, per_w)], idxs)
        pltpu.sync_copy(dst_hbm.at[pl.ds(base, per_w)], idxd)

        def start(j, rbuf_s, rbuf_d, sem_a, sem_b):
            pltpu.async_copy(tsrc_hbm.at[idxs.at[pl.ds(j * 128, 128)]], rbuf_s, sem_a)
            pltpu.async_copy(u3_hbm.at[idxd.at[pl.ds(j * 128, 128)]], rbuf_d, sem_b)

        def finish(j, rbuf_s, rbuf_d, sem_a, sem_b):
            pltpu.make_async_copy(tsrc_hbm.at[idxs.at[pl.ds(j * 128, 128)]], rbuf_s, sem_a).wait()
            pltpu.make_async_copy(u3_hbm.at[idxd.at[pl.ds(j * 128, 128)]], rbuf_d, sem_b).wait()
            pltpu.sync_copy(rbuf_s, gs_hbm.at[pl.ds(base + j * 128, 128)])
            pltpu.sync_copy(rbuf_d, gd_hbm.at[pl.ds(base + j * 128, 128)])

        start(0, rs0, rd0, sem_s0, sem_d0)

        def body(j2, carry):
            j = j2 * 2
            start(j + 1, rs1, rd1, sem_s1, sem_d1)
            finish(j, rs0, rd0, sem_s0, sem_d0)

            @pl.when(j + 2 < n_full)
            def _():
                start(j + 2, rs0, rd0, sem_s0, sem_d0)

            finish(j + 1, rs1, rd1, sem_s1, sem_d1)
            return carry

        lax.fori_loop(0, n_full // 2, body, 0)
        if tail:
            off = base + n_full * 128
            pltpu.sync_copy(src_hbm.at[pl.ds(off, tail)], idxs.at[pl.ds(0, tail)])
            pltpu.async_copy(tsrc_hbm.at[idxs.at[pl.ds(0, tail)]],
                             rs0.at[pl.ds(0, tail)], sem_s0).wait()
            pltpu.sync_copy(rs0.at[pl.ds(0, tail)], gs_hbm.at[pl.ds(off, tail)])
            pltpu.sync_copy(dst_hbm.at[pl.ds(off, tail)], idxd.at[pl.ds(0, tail)])
            pltpu.async_copy(u3_hbm.at[idxd.at[pl.ds(0, tail)]],
                             rd0.at[pl.ds(0, tail)], sem_d0).wait()
            pltpu.sync_copy(rd0.at[pl.ds(0, tail)], gd_hbm.at[pl.ds(off, tail)])

    return k(tsrc, u3, src, dst)


def _pass1_body(ea_ref, gs_ref, gd_ref, r_ref, cf_ref, wb_ref, bb_ref,
                wmid_ref, be1_ref, h1_ref, st_ref, acc):
    i = pl.program_id(0)
    a = jnp.dot(ea_ref[...], r_ref[...], preferred_element_type=F32)
    d = a - cf_ref[...]
    g = jnp.exp(-(d * d))
    g = _lrelu(jnp.dot(g, wb_ref[...], preferred_element_type=F32) + bb_ref[...])
    h1 = _lrelu(_unpack_lo(gs_ref[...]) + gd_ref[...] +
                jnp.dot(g, wmid_ref[...], preferred_element_type=F32) + be1_ref[...])
    h1_ref[...] = h1

    @pl.when(i == 0)
    def _():
        acc[...] = jnp.zeros_like(acc)

    acc[0:1, :] += jnp.sum(h1, axis=0, keepdims=True)
    acc[1:2, :] += jnp.sum(h1 * h1, axis=0, keepdims=True)

    @pl.when(i == pl.num_programs(0) - 1)
    def _():
        st_ref[...] = acc[...]


def _pass1(ea, gs, gd, rmat, cf, wb, bb, wmid, be1, eb):
    e = ea.shape[0]
    grid = (e // eb,)
    return pl.pallas_call(
        _pass1_body,
        grid=grid,
        in_specs=[
            pl.BlockSpec((eb, 4), lambda i: (i, 0)),
            pl.BlockSpec((eb, H), lambda i: (i, 0)),
            pl.BlockSpec((eb, H), lambda i: (i, 0)),
            pl.BlockSpec((4, 4 * NSTEP), lambda i: (0, 0)),
            pl.BlockSpec((1, 4 * NSTEP), lambda i: (0, 0)),
            pl.BlockSpec((4 * NSTEP, H), lambda i: (0, 0)),
            pl.BlockSpec((1, H), lambda i: (0, 0)),
            pl.BlockSpec((H, H), lambda i: (0, 0)),
            pl.BlockSpec((1, H), lambda i: (0, 0)),
        ],
        out_specs=[
            pl.BlockSpec((eb, H), lambda i: (i, 0)),
            pl.BlockSpec((8, H), lambda i: (0, 0)),
        ],
        out_shape=[
            jax.ShapeDtypeStruct((e, H), F32),
            jax.ShapeDtypeStruct((8, H), F32),
        ],
        scratch_shapes=[pltpu.VMEM((8, H), F32)],
    )(ea, gs, gd, rmat, cf, wb, bb, wmid, be1)


def _bn_vecs(st_ref, ne, g_ref, bt_ref):
    m = st_ref[0:1, :] * (1.0 / ne)
    var = st_ref[1:2, :] * (1.0 / ne) - m * m
    sc = g_ref[...] * lax.rsqrt(var + EPS)
    return m, sc, bt_ref[...]


def _stats2_body(h1_ref, st1_ref, ge1_ref, bte1_ref, we2_ref, be2_ref,
                 st2_ref, acc, *, ne):
    i = pl.program_id(0)
    m1, sc1, bt1 = _bn_vecs(st1_ref, ne, ge1_ref, bte1_ref)
    hn = (h1_ref[...] - m1) * sc1 + bt1
    h2 = _lrelu(jnp.dot(hn, we2_ref[...], preferred_element_type=F32) + be2_ref[...])

    @pl.when(i == 0)
    def _():
        acc[...] = jnp.zeros_like(acc)

    acc[0:1, :] += jnp.sum(h2, axis=0, keepdims=True)
    acc[1:2, :] += jnp.sum(h2 * h2, axis=0, keepdims=True)

    @pl.when(i == pl.num_programs(0) - 1)
    def _():
        st2_ref[...] = acc[...]


def _stats2(h1, st1, ge1, bte1, we2, be2, eb):
    e = h1.shape[0]
    grid = (e // eb,)
    return pl.pallas_call(
        functools.partial(_stats2_body, ne=e),
        grid=grid,
        in_specs=[
            pl.BlockSpec((eb, H), lambda i: (i, 0)),
            pl.BlockSpec((8, H), lambda i: (0, 0)),
            pl.BlockSpec((1, H), lambda i: (0, 0)),
            pl.BlockSpec((1, H), lambda i: (0, 0)),
            pl.BlockSpec((H, H), lambda i: (0, 0)),
            pl.BlockSpec((1, H), lambda i: (0, 0)),
        ],
        out_specs=pl.BlockSpec((8, H), lambda i: (0, 0)),
        out_shape=jax.ShapeDtypeStruct((8, H), F32),
        scratch_shapes=[pltpu.VMEM((8, H), F32)],
    )(h1, st1, ge1, bte1, we2, be2)


def _msgs_body(h1_ref, gxd_ref, ea_ref, st1_ref, ge1_ref, bte1_ref,
               we2_ref, be2_ref, st2_ref, ge2_ref, bte2_ref,
               we3_ref, be3_ref, m_ref, *, ne):
    m1, sc1, bt1 = _bn_vecs(st1_ref, ne, ge1_ref, bte1_ref)
    hn = (h1_ref[...] - m1) * sc1 + bt1
    h2 = _lrelu(jnp.dot(hn, we2_ref[...], preferred_element_type=F32) + be2_ref[...])
    m2, sc2, bt2 = _bn_vecs(st2_ref, ne, ge2_ref, bte2_ref)
    hn2 = (h2 - m2) * sc2 + bt2
    h3 = jnp.dot(hn2, we3_ref[...], preferred_element_type=F32) + be3_ref[...]
    coef = jnp.cos((np.pi / 2) * ea_ref[:, 3:4])
    m_ref[...] = coef * h3 * _unpack_hi(gxd_ref[...])


def _msgs(h1, gs, ea, st1, ge1, bte1, we2, be2, st2, ge2, bte2, we3, be3, eb):
    e = h1.shape[0]
    grid = (e // eb,)
    return pl.pallas_call(
        functools.partial(_msgs_body, ne=e),
        grid=grid,
        in_specs=[
            pl.BlockSpec((eb, H), lambda i: (i, 0)),
            pl.BlockSpec((eb, H), lambda i: (i, 0)),
            pl.BlockSpec((eb, 4), lambda i: (i, 0)),
            pl.BlockSpec((8, H), lambda i: (0, 0)),
            pl.BlockSpec((1, H), lambda i: (0, 0)),
            pl.BlockSpec((1, H), lambda i: (0, 0)),
            pl.BlockSpec((H, H), lambda i: (0, 0)),
            pl.BlockSpec((1, H), lambda i: (0, 0)),
            pl.BlockSpec((8, H), lambda i: (0, 0)),
            pl.BlockSpec((1, H), lambda i: (0, 0)),
            pl.BlockSpec((1, H), lambda i: (0, 0)),
            pl.BlockSpec((H, H), lambda i: (0, 0)),
            pl.BlockSpec((1, H), lambda i: (0, 0)),
        ],
        out_specs=pl.BlockSpec((eb, H), lambda i: (i, 0)),
        out_shape=jax.ShapeDtypeStruct((e, H), F32),
    )(h1, gs, ea, st1, ge1, bte1, we2, be2, st2, ge2, bte2, we3, be3)


def _sc_scatter(m, dst, zeros):
    e = m.shape[0]
    n = zeros.shape[0]
    per_w = e // SC_WORKERS
    n_full = per_w // 128
    tail = per_w - n_full * 128
    mesh = plsc.VectorSubcoreMesh(core_axis_name="c", subcore_axis_name="s")

    @functools.partial(
        pl.kernel,
        mesh=mesh,
        out_type=jax.ShapeDtypeStruct((SC_CORES, n, H), F32),
        scratch_types=[
            pltpu.VMEM((128,), jnp.int32),
            pltpu.VMEM((128,), jnp.int32),
            pltpu.VMEM((16,), jnp.int32),
            pltpu.VMEM((128, H), F32),
            pltpu.VMEM((128, H), F32),
            pltpu.VMEM_SHARED((n, H), F32),
            pltpu.SemaphoreType.DMA,
            pltpu.SemaphoreType.DMA,
            pltpu.SemaphoreType.DMA,
            pltpu.SemaphoreType.DMA,
        ],
    )
    def k(m_hbm, dst_hbm, z_hbm, out_hbm, idx0, idx1, idxt, rb0, rb1, accsh,
          si0, si1, sm0, sm1):
        c = lax.axis_index("c")
        s = lax.axis_index("s")
        r0 = s * 640
        ncp = jnp.where(s == SC_SUBCORES - 1, (n - 640 * (SC_SUBCORES - 1)) // 40,
                        640 // 40)

        def cp_init(j, carry):
            off = r0 + j * 40
            pltpu.sync_copy(z_hbm.at[pl.ds(off, 40)], accsh.at[pl.ds(off, 40)])
            return carry

        lax.fori_loop(0, ncp, cp_init, 0)
        base = (c * SC_SUBCORES + s) * per_w
        plsc.subcore_barrier()

        def start(j, idxb, rbuf, semi, semm):
            pltpu.async_copy(dst_hbm.at[pl.ds(base + j * 128, 128)], idxb, semi)
            pltpu.async_copy(m_hbm.at[pl.ds(base + j * 128, 128)], rbuf, semm)

        def finish(j, idxb, rbuf, semi, semm):
            pltpu.make_async_copy(dst_hbm.at[pl.ds(base + j * 128, 128)], idxb, semi).wait()
            pltpu.make_async_copy(m_hbm.at[pl.ds(base + j * 128, 128)], rbuf, semm).wait()
            pltpu.sync_copy(rbuf, accsh.at[idxb], add=True)

        start(0, idx0, rb0, si0, sm0)

        def body(j2, carry):
            j = j2 * 2
            start(j + 1, idx1, rb1, si1, sm1)
            finish(j, idx0, rb0, si0, sm0)

            @pl.when(j + 2 < n_full)
            def _():
                start(j + 2, idx0, rb0, si0, sm0)

            finish(j + 1, idx1, rb1, si1, sm1)
            return carry

        lax.fori_loop(0, n_full // 2, body, 0)
        if tail:
            off = base + n_full * 128
            pltpu.sync_copy(dst_hbm.at[pl.ds(off, tail)], idxt)
            pltpu.sync_copy(m_hbm.at[pl.ds(off, tail)], rb0.at[pl.ds(0, tail)])
            pltpu.sync_copy(rb0.at[pl.ds(0, tail)], accsh.at[idxt], add=True)
        plsc.subcore_barrier()

        def cp_out(j, carry):
            off = r0 + j * 40
            pltpu.sync_copy(accsh.at[pl.ds(off, 40)], out_hbm.at[c, pl.ds(off, 40)])
            return carry

        lax.fori_loop(0, ncp, cp_out, 0)

    return k(m, dst, zeros)


def _node_stats_body(xd_ref, i0_ref, i1_ref, v_ref, wn1_ref, bn1_ref,
                     st_ref, acc):
    i = pl.program_id(0)
    z0 = v_ref[...] * xd_ref[...] + i0_ref[0] + i1_ref[0]
    n1 = _lrelu(jnp.dot(z0, wn1_ref[...], preferred_element_type=F32) + bn1_ref[...])

    @pl.when(i == 0)
    def _():
        acc[...] = jnp.zeros_like(acc)

    acc[0:1, :] += jnp.sum(n1, axis=0, keepdims=True)
    acc[1:2, :] += jnp.sum(n1 * n1, axis=0, keepdims=True)

    @pl.when(i == pl.num_programs(0) - 1)
    def _():
        st_ref[...] = acc[...]


def _node_apply_body(xd_ref, i0_ref, i1_ref, x_ref, st_ref, v_ref, wn1_ref,
                     bn1_ref, gn1_ref, btn1_ref, wn2_ref, bn2_ref, z_ref, *, nn):
    z0 = v_ref[...] * xd_ref[...] + i0_ref[0] + i1_ref[0]
    n1 = _lrelu(jnp.dot(z0, wn1_ref[...], preferred_element_type=F32) + bn1_ref[...])
    m, sc, bt = _bn_vecs(st_ref, nn, gn1_ref, btn1_ref)
    nbn = (n1 - m) * sc + bt
    z_ref[...] = (jnp.dot(nbn, wn2_ref[...], preferred_element_type=F32)
                  + bn2_ref[...] + x_ref[...])


def _node(xd, inc, x, v, wn1, bn1, gn1, btn1, wn2, bn2, nblk):
    n = x.shape[0]
    grid = (n // nblk,)
    blk = lambda shape: pl.BlockSpec(shape, lambda i: (0,) * len(shape))
    nodeb = pl.BlockSpec((nblk, H), lambda i: (i, 0))
    inc0 = pl.BlockSpec((1, nblk, H), lambda i: (0, i, 0))
    inc1 = pl.BlockSpec((1, nblk, H), lambda i: (1, i, 0))
    st = pl.pallas_call(
        _node_stats_body,
        grid=grid,
        in_specs=[nodeb, inc0, inc1, blk((1, H)), blk((H, H)), blk((1, H))],
        out_specs=pl.BlockSpec((8, H), lambda i: (0, 0)),
        out_shape=jax.ShapeDtypeStruct((8, H), F32),
        scratch_shapes=[pltpu.VMEM((8, H), F32)],
    )(xd, inc, inc, v, wn1, bn1)
    return pl.pallas_call(
        functools.partial(_node_apply_body, nn=n),
        grid=grid,
        in_specs=[nodeb, inc0, inc1, nodeb, blk((8, H)), blk((1, H)),
                  blk((H, H)), blk((1, H)), blk((1, H)), blk((1, H)),
                  blk((H, H)), blk((1, H))],
        out_specs=nodeb,
        out_shape=jax.ShapeDtypeStruct((n, H), F32),
    )(xd, inc, inc, x, st, v, wn1, bn1, gn1, btn1, wn2, bn2)


def kernel(x, edge_attr, edge_index, Wb, bb, w_e1, b_e1, g_e1, bt_e1,
           w_e2, b_e2, g_e2, bt_e2, w_e3, b_e3, w_d, b_d, v,
           w_n1, b_n1, g_n1, bt_n1, w_n2, b_n2):
    n = x.shape[0]
    e = edge_attr.shape[0]
    src = edge_index[0]
    dst = edge_index[1]

    w1a = w_e1[0:H]
    wmid = w_e1[H:2 * H]
    w1c = w_e1[2 * H:3 * H]

    row = lambda b: b.reshape(1, -1)

    rmat = jnp.repeat(jnp.eye(4, dtype=F32) * NSTEP, NSTEP, axis=1)
    cf = (jnp.tile(jnp.linspace(0.0, 1.0, NSTEP), 4) * NSTEP).astype(F32).reshape(1, -1)

    tsrc, u3, xd = _prep(x, w1a, w1c, w_d, row(b_d), nblk=2000)
    gs, gd = _sc_gather(tsrc, u3, src, dst)

    eb = 2560
    h1, st1 = _pass1(edge_attr, gs, gd, rmat, cf, Wb, row(bb), wmid,
                     row(b_e1), eb)
    st2 = _stats2(h1, st1, row(g_e1), row(bt_e1), w_e2, row(b_e2), eb)
    m = _msgs(h1, gs, edge_attr, st1, row(g_e1), row(bt_e1), w_e2, row(b_e2),
              st2, row(g_e2), row(bt_e2), w_e3, row(b_e3), eb)

    zeros = jnp.zeros((n, H), F32)
    inc = _sc_scatter(m, dst, zeros)

    return _node(xd, inc, x, v, w_n1, row(b_n1), g_n1.reshape(1, -1),
                 bt_n1.reshape(1, -1), w_n2, row(b_n2), nblk=2000)


# speedup vs baseline: 1.2708x; 1.0455x over previous
import functools

import numpy as np
import jax
import jax.numpy as jnp
from jax import lax
from jax.experimental import pallas as pl
from jax.experimental.pallas import tpu as pltpu
from jax.experimental.pallas import tpu_sc as plsc

F32 = jnp.float32
BF16 = jnp.bfloat16
H = 128
NSTEP = 50
EPS = 1e-5

SC_CORES = 2
SC_SUBCORES = 16
SC_WORKERS = SC_CORES * SC_SUBCORES


def _lrelu(x):
    return jnp.where(x >= 0, x, 0.01 * x)


_MASK_HI = -65536
_MASK_LO = 0xFFFF


def _rne_bf16_bits(f):
    b = lax.bitcast_convert_type(f, jnp.int32)
    return b + 0x7FFF + ((b >> 16) & 1)


def _pack_pair(lo, hi):
    return (((_rne_bf16_bits(lo) >> 16) & _MASK_LO)
            | (_rne_bf16_bits(hi) & _MASK_HI))


def _unpack_lo(p):
    return lax.bitcast_convert_type(p << 16, F32)


def _unpack_hi(p):
    return lax.bitcast_convert_type(p & _MASK_HI, F32)


def _prep_body(x_ref, w1a_ref, w1c_ref, wd_ref, bd_ref, tsrc_ref, u3_ref,
               xd_ref):
    xb = x_ref[...]
    xd = jnp.dot(xb, wd_ref[...], preferred_element_type=F32) + bd_ref[...]
    u1 = jnp.dot(xb, w1a_ref[...], preferred_element_type=F32)
    tsrc_ref[...] = _pack_pair(u1, xd)
    u3_ref[...] = jnp.dot(xb, w1c_ref[...], preferred_element_type=F32)
    xd_ref[...] = xd


def _prep(x, w1a, w1c, wd, bd, nblk):
    n = x.shape[0]
    grid = (n // nblk,)
    return pl.pallas_call(
        _prep_body,
        grid=grid,
        in_specs=[
            pl.BlockSpec((nblk, H), lambda i: (i, 0)),
            pl.BlockSpec((H, H), lambda i: (0, 0)),
            pl.BlockSpec((H, H), lambda i: (0, 0)),
            pl.BlockSpec((H, H), lambda i: (0, 0)),
            pl.BlockSpec((1, H), lambda i: (0, 0)),
        ],
        out_specs=[
            pl.BlockSpec((nblk, H), lambda i: (i, 0)),
            pl.BlockSpec((nblk, H), lambda i: (i, 0)),
            pl.BlockSpec((nblk, H), lambda i: (i, 0)),
        ],
        out_shape=[
            jax.ShapeDtypeStruct((n, H), jnp.int32),
            jax.ShapeDtypeStruct((n, H), F32),
            jax.ShapeDtypeStruct((n, H), F32),
        ],
    )(x, w1a, w1c, wd, bd)


def _sc_gather(tsrc, u3, src, dst):
    e = src.shape[0]
    per_w = e // SC_WORKERS
    n_full = per_w // 128
    tail = per_w - n_full * 128
    mesh = plsc.VectorSubcoreMesh(core_axis_name="c", subcore_axis_name="s")

    @functools.partial(
        pl.kernel,
        mesh=mesh,
        out_type=[
            jax.ShapeDtypeStruct((e, H), jnp.int32),
            jax.ShapeDtypeStruct((e, H), F32),
        ],
        scratch_types=[
            pltpu.VMEM((per_w,), jnp.int32),
            pltpu.VMEM((per_w,), jnp.int32),
            pltpu.VMEM((128, H), jnp.int32),
            pltpu.VMEM((128, H), jnp.int32),
            pltpu.VMEM((128, H), F32),
            pltpu.VMEM((128, H), F32),
            pltpu.SemaphoreType.DMA,
            pltpu.SemaphoreType.DMA,
            pltpu.SemaphoreType.DMA,
            pltpu.SemaphoreType.DMA,
        ],
    )
    def k(tsrc_hbm, u3_hbm, src_hbm, dst_hbm, gs_hbm, gd_hbm,
          idxs, idxd, rs0, rs1, rd0, rd1, sem_s0, sem_s1, sem_d0, sem_d1):
        c = lax.axis_index("c")
        s = lax.axis_index("s")
        wid = s * SC_CORES + c
        base = wid * per_w

        pltpu.sync_copy(src_hbm.at[pl.ds(base, per_w)], idxs)
        pltpu.sync_copy(dst_hbm.at[pl.ds(base, per_w)], idxd)

        def start(j, rbuf_s, rbuf_d, sem_a, sem_b):
            pltpu.async_copy(tsrc_hbm.at[idxs.at[pl.ds(j * 128, 128)]], rbuf_s, sem_a)
            pltpu.async_copy(u3_hbm.at[idxd.at[pl.ds(j * 128, 128)]], rbuf_d, sem_b)

        def finish(j, rbuf_s, rbuf_d, sem_a, sem_b):
            pltpu.make_async_copy(tsrc_hbm.at[idxs.at[pl.ds(j * 128, 128)]], rbuf_s, sem_a).wait()
            pltpu.make_async_copy(u3_hbm.at[idxd.at[pl.ds(j * 128, 128)]], rbuf_d, sem_b).wait()
            pltpu.sync_copy(rbuf_s, gs_hbm.at[pl.ds(base + j * 128, 128)])
            pltpu.sync_copy(rbuf_d, gd_hbm.at[pl.ds(base + j * 128, 128)])

        start(0, rs0, rd0, sem_s0, sem_d0)

        def body(j2, carry):
            j = j2 * 2
            start(j + 1, rs1, rd1, sem_s1, sem_d1)
            finish(j, rs0, rd0, sem_s0, sem_d0)

            @pl.when(j + 2 < n_full)
            def _():
                start(j + 2, rs0, rd0, sem_s0, sem_d0)

            finish(j + 1, rs1, rd1, sem_s1, sem_d1)
            return carry

        lax.fori_loop(0, n_full // 2, body, 0)
        if tail:
            off = base + n_full * 128
            pltpu.sync_copy(src_hbm.at[pl.ds(off, tail)], idxs.at[pl.ds(0, tail)])
            pltpu.async_copy(tsrc_hbm.at[idxs.at[pl.ds(0, tail)]],
                             rs0.at[pl.ds(0, tail)], sem_s0).wait()
            pltpu.sync_copy(rs0.at[pl.ds(0, tail)], gs_hbm.at[pl.ds(off, tail)])
            pltpu.sync_copy(dst_hbm.at[pl.ds(off, tail)], idxd.at[pl.ds(0, tail)])
            pltpu.async_copy(u3_hbm.at[idxd.at[pl.ds(0, tail)]],
                             rd0.at[pl.ds(0, tail)], sem_d0).wait()
            pltpu.sync_copy(rd0.at[pl.ds(0, tail)], gd_hbm.at[pl.ds(off, tail)])

    return k(tsrc, u3, src, dst)


_DN0 = (((0,), (0,)), ((), ()))


def _pass1_body(ea_ref, gs_ref, gd_ref, r_ref, cf_ref, wb_ref, bb_ref,
                wmid_ref, be1_ref, h1_ref, st_ref, acc):
    i = pl.program_id(0)
    a = lax.dot_general(ea_ref[...], r_ref[...], _DN0,
                        preferred_element_type=F32)
    d = a - cf_ref[...]
    g = jnp.exp(-(d * d))
    g = _lrelu(jnp.dot(g, wb_ref[...], preferred_element_type=F32) + bb_ref[...])
    h1 = _lrelu(_unpack_lo(gs_ref[...]) + gd_ref[...] +
                jnp.dot(g, wmid_ref[...], preferred_element_type=F32) + be1_ref[...])
    h1_ref[...] = h1

    @pl.when(i == 0)
    def _():
        acc[...] = jnp.zeros_like(acc)

    acc[0:1, :] += jnp.sum(h1, axis=0, keepdims=True)
    acc[1:2, :] += jnp.sum(h1 * h1, axis=0, keepdims=True)

    @pl.when(i == pl.num_programs(0) - 1)
    def _():
        st_ref[...] = acc[...]


def _pass1(ea, gs, gd, rmat, cf, wb, bb, wmid, be1, eb):
    e = gs.shape[0]
    grid = (e // eb,)
    return pl.pallas_call(
        _pass1_body,
        grid=grid,
        in_specs=[
            pl.BlockSpec((4, eb), lambda i: (0, i)),
            pl.BlockSpec((eb, H), lambda i: (i, 0)),
            pl.BlockSpec((eb, H), lambda i: (i, 0)),
            pl.BlockSpec((4, 4 * NSTEP), lambda i: (0, 0)),
            pl.BlockSpec((1, 4 * NSTEP), lambda i: (0, 0)),
            pl.BlockSpec((4 * NSTEP, H), lambda i: (0, 0)),
            pl.BlockSpec((1, H), lambda i: (0, 0)),
            pl.BlockSpec((H, H), lambda i: (0, 0)),
            pl.BlockSpec((1, H), lambda i: (0, 0)),
        ],
        out_specs=[
            pl.BlockSpec((eb, H), lambda i: (i, 0)),
            pl.BlockSpec((8, H), lambda i: (0, 0)),
        ],
        out_shape=[
            jax.ShapeDtypeStruct((e, H), F32),
            jax.ShapeDtypeStruct((8, H), F32),
        ],
        scratch_shapes=[pltpu.VMEM((8, H), F32)],
    )(ea, gs, gd, rmat, cf, wb, bb, wmid, be1)


def _bn_fold(st, ne, g, bt, w_next, b_next):
    mean = st[0] * (1.0 / ne)
    var = st[1] * (1.0 / ne) - mean * mean
    a = g * lax.rsqrt(var + EPS)
    weff = w_next * a[:, None]
    beff = ((bt - mean * a) @ w_next + b_next).reshape(1, -1)
    return weff, beff


def _stats2_body(h1_ref, w2_ref, b2_ref, st2_ref, acc):
    i = pl.program_id(0)
    h2 = _lrelu(jnp.dot(h1_ref[...], w2_ref[...], preferred_element_type=F32)
                + b2_ref[...])

    @pl.when(i == 0)
    def _():
        acc[...] = jnp.zeros_like(acc)

    acc[0:1, :] += jnp.sum(h2, axis=0, keepdims=True)
    acc[1:2, :] += jnp.sum(h2 * h2, axis=0, keepdims=True)

    @pl.when(i == pl.num_programs(0) - 1)
    def _():
        st2_ref[...] = acc[...]


def _stats2(h1, w2eff, b2eff, eb):
    e = h1.shape[0]
    grid = (e // eb,)
    return pl.pallas_call(
        _stats2_body,
        grid=grid,
        in_specs=[
            pl.BlockSpec((eb, H), lambda i: (i, 0)),
            pl.BlockSpec((H, H), lambda i: (0, 0)),
            pl.BlockSpec((1, H), lambda i: (0, 0)),
        ],
        out_specs=pl.BlockSpec((8, H), lambda i: (0, 0)),
        out_shape=jax.ShapeDtypeStruct((8, H), F32),
        scratch_shapes=[pltpu.VMEM((8, H), F32)],
    )(h1, w2eff, b2eff)


def _msgs_body(h1_ref, gxd_ref, ea_ref, sel_ref, w2_ref, b2_ref,
               w3_ref, b3_ref, m_ref):
    h2 = _lrelu(jnp.dot(h1_ref[...], w2_ref[...], preferred_element_type=F32)
                + b2_ref[...])
    h3 = jnp.dot(h2, w3_ref[...], preferred_element_type=F32) + b3_ref[...]
    ea3 = lax.dot_general(ea_ref[...], sel_ref[...], _DN0,
                          preferred_element_type=F32)
    coef = jnp.cos((np.pi / 2) * ea3[:, 0:1])
    m_ref[...] = coef * h3 * _unpack_hi(gxd_ref[...])


def _msgs(h1, gs, eat, sel3, w2eff, b2eff, w3eff, b3eff, eb):
    e = h1.shape[0]
    grid = (e // eb,)
    return pl.pallas_call(
        _msgs_body,
        grid=grid,
        in_specs=[
            pl.BlockSpec((eb, H), lambda i: (i, 0)),
            pl.BlockSpec((eb, H), lambda i: (i, 0)),
            pl.BlockSpec((4, eb), lambda i: (0, i)),
            pl.BlockSpec((4, 8), lambda i: (0, 0)),
            pl.BlockSpec((H, H), lambda i: (0, 0)),
            pl.BlockSpec((1, H), lambda i: (0, 0)),
            pl.BlockSpec((H, H), lambda i: (0, 0)),
            pl.BlockSpec((1, H), lambda i: (0, 0)),
        ],
        out_specs=pl.BlockSpec((eb, H), lambda i: (i, 0)),
        out_shape=jax.ShapeDtypeStruct((e, H), F32),
    )(h1, gs, eat, sel3, w2eff, b2eff, w3eff, b3eff)


def _sc_scatter(m, dst, zeros):
    e = m.shape[0]
    n = zeros.shape[0]
    per_w = e // SC_WORKERS
    n_full = per_w // 128
    tail = per_w - n_full * 128
    mesh = plsc.VectorSubcoreMesh(core_axis_name="c", subcore_axis_name="s")

    @functools.partial(
        pl.kernel,
        mesh=mesh,
        out_type=jax.ShapeDtypeStruct((SC_CORES, n, H), F32),
        scratch_types=[
            pltpu.VMEM((128,), jnp.int32),
            pltpu.VMEM((128,), jnp.int32),
            pltpu.VMEM((16,), jnp.int32),
            pltpu.VMEM((128, H), F32),
            pltpu.VMEM((128, H), F32),
            pltpu.VMEM_SHARED((n, H), F32),
            pltpu.SemaphoreType.DMA,
            pltpu.SemaphoreType.DMA,
            pltpu.SemaphoreType.DMA,
            pltpu.SemaphoreType.DMA,
        ],
    )
    def k(m_hbm, dst_hbm, z_hbm, out_hbm, idx0, idx1, idxt, rb0, rb1, accsh,
          si0, si1, sm0, sm1):
        c = lax.axis_index("c")
        s = lax.axis_index("s")
        r0 = s * 640
        ncp = jnp.where(s == SC_SUBCORES - 1, (n - 640 * (SC_SUBCORES - 1)) // 40,
                        640 // 40)

        def cp_init(j, carry):
            off = r0 + j * 40
            pltpu.sync_copy(z_hbm.at[pl.ds(off, 40)], accsh.at[pl.ds(off, 40)])
            return carry

        lax.fori_loop(0, ncp, cp_init, 0)
        base = (c * SC_SUBCORES + s) * per_w
        plsc.subcore_barrier()

        def start(j, idxb, rbuf, semi, semm):
            pltpu.async_copy(dst_hbm.at[pl.ds(base + j * 128, 128)], idxb, semi)
            pltpu.async_copy(m_hbm.at[pl.ds(base + j * 128, 128)], rbuf, semm)

        def finish(j, idxb, rbuf, semi, semm):
            pltpu.make_async_copy(dst_hbm.at[pl.ds(base + j * 128, 128)], idxb, semi).wait()
            pltpu.make_async_copy(m_hbm.at[pl.ds(base + j * 128, 128)], rbuf, semm).wait()
            pltpu.sync_copy(rbuf, accsh.at[idxb], add=True)

        start(0, idx0, rb0, si0, sm0)

        def body(j2, carry):
            j = j2 * 2
            start(j + 1, idx1, rb1, si1, sm1)
            finish(j, idx0, rb0, si0, sm0)

            @pl.when(j + 2 < n_full)
            def _():
                start(j + 2, idx0, rb0, si0, sm0)

            finish(j + 1, idx1, rb1, si1, sm1)
            return carry

        lax.fori_loop(0, n_full // 2, body, 0)
        if tail:
            off = base + n_full * 128
            pltpu.sync_copy(dst_hbm.at[pl.ds(off, tail)], idxt)
            pltpu.sync_copy(m_hbm.at[pl.ds(off, tail)], rb0.at[pl.ds(0, tail)])
            pltpu.sync_copy(rb0.at[pl.ds(0, tail)], accsh.at[idxt], add=True)
        plsc.subcore_barrier()

        def cp_out(j, carry):
            off = r0 + j * 40
            pltpu.sync_copy(accsh.at[pl.ds(off, 40)], out_hbm.at[c, pl.ds(off, 40)])
            return carry

        lax.fori_loop(0, ncp, cp_out, 0)

    return k(m, dst, zeros)


def _node_stats_body(xd_ref, i0_ref, i1_ref, v_ref, wn1_ref, bn1_ref,
                     st_ref, acc):
    i = pl.program_id(0)
    z0 = v_ref[...] * xd_ref[...] + i0_ref[0] + i1_ref[0]
    n1 = _lrelu(jnp.dot(z0, wn1_ref[...], preferred_element_type=F32) + bn1_ref[...])

    @pl.when(i == 0)
    def _():
        acc[...] = jnp.zeros_like(acc)

    acc[0:1, :] += jnp.sum(n1, axis=0, keepdims=True)
    acc[1:2, :] += jnp.sum(n1 * n1, axis=0, keepdims=True)

    @pl.when(i == pl.num_programs(0) - 1)
    def _():
        st_ref[...] = acc[...]


def _node_apply_body(xd_ref, i0_ref, i1_ref, x_ref, v_ref, wn1_ref,
                     bn1_ref, wn2_ref, bn2_ref, z_ref):
    z0 = v_ref[...] * xd_ref[...] + i0_ref[0] + i1_ref[0]
    n1 = _lrelu(jnp.dot(z0, wn1_ref[...], preferred_element_type=F32) + bn1_ref[...])
    z_ref[...] = (jnp.dot(n1, wn2_ref[...], preferred_element_type=F32)
                  + bn2_ref[...] + x_ref[...])


def _node_stats(xd, inc, v, wn1, bn1, nblk):
    n = xd.shape[0]
    grid = (n // nblk,)
    blk = lambda shape: pl.BlockSpec(shape, lambda i: (0,) * len(shape))
    nodeb = pl.BlockSpec((nblk, H), lambda i: (i, 0))
    inc0 = pl.BlockSpec((1, nblk, H), lambda i: (0, i, 0))
    inc1 = pl.BlockSpec((1, nblk, H), lambda i: (1, i, 0))
    return pl.pallas_call(
        _node_stats_body,
        grid=grid,
        in_specs=[nodeb, inc0, inc1, blk((1, H)), blk((H, H)), blk((1, H))],
        out_specs=pl.BlockSpec((8, H), lambda i: (0, 0)),
        out_shape=jax.ShapeDtypeStruct((8, H), F32),
        scratch_shapes=[pltpu.VMEM((8, H), F32)],
    )(xd, inc, inc, v, wn1, bn1)


def _node_apply(xd, inc, x, v, wn1, bn1, wn2eff, bn2eff, nblk):
    n = x.shape[0]
    grid = (n // nblk,)
    blk = lambda shape: pl.BlockSpec(shape, lambda i: (0,) * len(shape))
    nodeb = pl.BlockSpec((nblk, H), lambda i: (i, 0))
    inc0 = pl.BlockSpec((1, nblk, H), lambda i: (0, i, 0))
    inc1 = pl.BlockSpec((1, nblk, H), lambda i: (1, i, 0))
    return pl.pallas_call(
        _node_apply_body,
        grid=grid,
        in_specs=[nodeb, inc0, inc1, nodeb, blk((1, H)), blk((H, H)),
                  blk((1, H)), blk((H, H)), blk((1, H))],
        out_specs=nodeb,
        out_shape=jax.ShapeDtypeStruct((n, H), F32),
    )(xd, inc, inc, x, v, wn1, bn1, wn2eff, bn2eff)


def kernel(x, edge_attr, edge_index, Wb, bb, w_e1, b_e1, g_e1, bt_e1,
           w_e2, b_e2, g_e2, bt_e2, w_e3, b_e3, w_d, b_d, v,
           w_n1, b_n1, g_n1, bt_n1, w_n2, b_n2):
    n = x.shape[0]
    e = edge_attr.shape[0]
    src = edge_index[0]
    dst = edge_index[1]

    w1a = w_e1[0:H]
    wmid = w_e1[H:2 * H]
    w1c = w_e1[2 * H:3 * H]

    row = lambda b: b.reshape(1, -1)

    rmat = jnp.repeat(jnp.eye(4, dtype=F32) * NSTEP, NSTEP, axis=1)
    cf = (jnp.tile(jnp.linspace(0.0, 1.0, NSTEP), 4) * NSTEP).astype(F32).reshape(1, -1)

    tsrc, u3, xd = _prep(x, w1a, w1c, w_d, row(b_d), nblk=2000)
    gs, gd = _sc_gather(tsrc, u3, src, dst)

    eat = edge_attr.T
    sel3 = jnp.zeros((4, 8), F32).at[3, 0].set(1.0)

    eb = 2560
    h1, st1 = _pass1(eat, gs, gd, rmat, cf, Wb, row(bb), wmid, row(b_e1), eb)
    w2eff, b2eff = _bn_fold(st1, e, g_e1, bt_e1, w_e2, b_e2)
    st2 = _stats2(h1, w2eff, b2eff, eb)
    w3eff, b3eff = _bn_fold(st2, e, g_e2, bt_e2, w_e3, b_e3)
    m = _msgs(h1, gs, eat, sel3, w2eff, b2eff, w3eff, b3eff, eb)

    zeros = jnp.zeros((n, H), F32)
    inc = _sc_scatter(m, dst, zeros)

    stn = _node_stats(xd, inc, v, w_n1, row(b_n1), nblk=2000)
    wn2eff, bn2eff = _bn_fold(stn, n, g_n1, bt_n1, w_n2, b_n2)
    return _node_apply(xd, inc, x, v, w_n1, row(b_n1), wn2eff, bn2eff,
                       nblk=2000)
